# baseline (speedup 1.0000x reference)
import functools

import jax
import jax.numpy as jnp
from jax.experimental import pallas as pl

_E = 16
_K = 2
_D = 2048
_TB = 1024


def _router_block(x_ref, wt_ref, b_ref, logits_ref, probs_ref, wts_ref, idx_ref):
    x = x_ref[...]
    logits = jnp.dot(x, wt_ref[...],
                     preferred_element_type=jnp.float32) + b_ref[...]
    logits_ref[...] = logits

    m = jnp.max(logits, axis=-1, keepdims=True)
    e = jnp.exp(logits - m)
    probs = e / jnp.sum(e, axis=-1, keepdims=True)
    probs_ref[...] = probs

    iota = jax.lax.broadcasted_iota(jnp.int32, probs.shape, 1)
    m1 = jnp.max(probs, axis=-1, keepdims=True)
    i1 = jnp.min(jnp.where(probs == m1, iota, _E), axis=-1, keepdims=True)
    masked = jnp.where(iota == i1, -jnp.inf, probs)
    m2 = jnp.max(masked, axis=-1, keepdims=True)
    i2 = jnp.min(jnp.where(masked == m2, iota, _E), axis=-1, keepdims=True)
    s = m1 + m2
    wts_ref[...] = jnp.concatenate([m1 / s, m2 / s], axis=-1)
    idx_ref[...] = jnp.concatenate([i1, i2], axis=-1)


@functools.partial(jax.jit, static_argnames=("interpret",))
def kernel(inputs, W, b, interpret=False):
    B, S, D = inputs.shape
    T = B * S
    x = inputs.reshape(T, D)
    wt = W.T
    b2 = b.reshape(1, _E)

    grid = (T // _TB,)
    out = pl.pallas_call(
        _router_block,
        grid=grid,
        in_specs=[
            pl.BlockSpec((_TB, D), lambda i: (i, 0)),
            pl.BlockSpec((D, _E), lambda i: (0, 0)),
            pl.BlockSpec((1, _E), lambda i: (0, 0)),
        ],
        out_specs=[
            pl.BlockSpec((_TB, _E), lambda i: (i, 0)),
            pl.BlockSpec((_TB, _E), lambda i: (i, 0)),
            pl.BlockSpec((_TB, _K), lambda i: (i, 0)),
            pl.BlockSpec((_TB, _K), lambda i: (i, 0)),
        ],
        out_shape=[
            jax.ShapeDtypeStruct((T, _E), jnp.float32),
            jax.ShapeDtypeStruct((T, _E), jnp.float32),
            jax.ShapeDtypeStruct((T, _K), jnp.float32),
            jax.ShapeDtypeStruct((T, _K), jnp.int32),
        ],
        interpret=interpret,
    )(x, wt, b2)

    logits, probs, wts, idx = out
    return (logits.reshape(B, S, _E), probs.reshape(B, S, _E),
            wts.reshape(B, S, _K), idx.reshape(B, S, _K))


# speedup vs baseline: 1.0187x; 1.0187x over previous
import functools

import jax
import jax.numpy as jnp
from jax.experimental import pallas as pl
from jax.experimental.pallas import tpu as pltpu

_E = 16
_K = 2
_D = 2048
_TB = 1024


def _router_block(x_ref, wt_ref, b_ref, logits_ref, probs_ref, wts_ref, idx_ref):
    x = x_ref[...]
    logits = jnp.dot(x, wt_ref[...],
                     preferred_element_type=jnp.float32) + b_ref[...]
    logits_ref[...] = logits

    iota = jax.lax.broadcasted_iota(jnp.int32, logits.shape, 1)
    m1 = jnp.max(logits, axis=-1, keepdims=True)
    i1 = jnp.min(jnp.where(logits == m1, iota, _E), axis=-1, keepdims=True)
    masked = jnp.where(iota == i1, -jnp.inf, logits)
    m2 = jnp.max(masked, axis=-1, keepdims=True)
    i2 = jnp.min(jnp.where(masked == m2, iota, _E), axis=-1, keepdims=True)

    e = jnp.exp(logits - m1)
    probs_ref[...] = e / jnp.sum(e, axis=-1, keepdims=True)

    e2 = jnp.exp(m2 - m1)
    w1 = 1.0 / (1.0 + e2)
    wts_ref[...] = jnp.concatenate([w1, 1.0 - w1], axis=-1)
    idx_ref[...] = jnp.concatenate([i1, i2], axis=-1)


@functools.partial(jax.jit, static_argnames=("interpret",))
def kernel(inputs, W, b, interpret=False):
    B, S, D = inputs.shape
    T = B * S
    x = inputs.reshape(T, D)
    wt = W.T
    b2 = b.reshape(1, _E)

    grid = (T // _TB,)
    out = pl.pallas_call(
        _router_block,
        grid=grid,
        in_specs=[
            pl.BlockSpec((_TB, D), lambda i: (i, 0)),
            pl.BlockSpec((D, _E), lambda i: (0, 0)),
            pl.BlockSpec((1, _E), lambda i: (0, 0)),
        ],
        out_specs=[
            pl.BlockSpec((_TB, _E), lambda i: (i, 0)),
            pl.BlockSpec((_TB, _E), lambda i: (i, 0)),
            pl.BlockSpec((_TB, _K), lambda i: (i, 0)),
            pl.BlockSpec((_TB, _K), lambda i: (i, 0)),
        ],
        out_shape=[
            jax.ShapeDtypeStruct((T, _E), jnp.float32),
            jax.ShapeDtypeStruct((T, _E), jnp.float32),
            jax.ShapeDtypeStruct((T, _K), jnp.float32),
            jax.ShapeDtypeStruct((T, _K), jnp.int32),
        ],
        compiler_params=pltpu.CompilerParams(
            dimension_semantics=("parallel",),
        ),
        interpret=interpret,
    )(x, wt, b2)

    logits, probs, wts, idx = out
    return (logits.reshape(B, S, _E), probs.reshape(B, S, _E),
            wts.reshape(B, S, _K), idx.reshape(B, S, _K))
